# Initial kernel scaffold; baseline (speedup 1.0000x reference)
import jax
import jax.numpy as jnp
from jax.experimental import pallas as pl


def kernel(node_embeddings, edge_index):
    raise NotImplementedError("write your pallas kernel here")


# speedup vs baseline: 2.6467x; 2.6467x over previous
import functools

import jax
import jax.numpy as jnp
from jax import lax
from jax.experimental import pallas as pl
from jax.experimental.pallas import tpu as pltpu
from jax.experimental.pallas import tpu_sc as plsc

N_NODES = 10000
D = 128
E = 320000
NC = 2
NS = 16
NW = NC * NS
E_PER_W = E // NW
C = 80
NCHUNKS = E_PER_W // C
L = 16

_mesh = plsc.VectorSubcoreMesh(core_axis_name="c", subcore_axis_name="s")


@functools.partial(
    pl.kernel,
    out_type=jax.ShapeDtypeStruct((E,), jnp.float32),
    mesh=_mesh,
    compiler_params=pltpu.CompilerParams(needs_layout_passes=False),
    scratch_types=[
        pltpu.VMEM((E_PER_W,), jnp.int32),
        pltpu.VMEM((E_PER_W,), jnp.int32),
        pltpu.VMEM((C, D), jnp.float32),
        pltpu.VMEM((C, D), jnp.float32),
        pltpu.VMEM((E_PER_W,), jnp.float32),
    ],
)
def _edge_dot(table_hbm, sidx_hbm, tidx_hbm, out_hbm,
              sidx_v, tidx_v, srows_v, trows_v, out_v):
    wid = lax.axis_index("s") * NC + lax.axis_index("c")
    base = wid * E_PER_W

    pltpu.sync_copy(sidx_hbm.at[pl.ds(base, E_PER_W)], sidx_v)
    pltpu.sync_copy(tidx_hbm.at[pl.ds(base, E_PER_W)], tidx_v)

    lane = lax.iota(jnp.int32, L)

    @pl.loop(0, NCHUNKS)
    def _(ch):
        coff = ch * C
        pltpu.sync_copy(table_hbm.at[sidx_v.at[pl.ds(coff, C)]], srows_v)
        pltpu.sync_copy(table_hbm.at[tidx_v.at[pl.ds(coff, C)]], trows_v)

        @pl.loop(0, C, step=L)
        def _(e0):
            res = jnp.zeros((L,), jnp.float32)
            for e in range(L):
                r = e0 + e
                acc = srows_v[r, pl.ds(0, L)] * trows_v[r, pl.ds(0, L)]
                for j in range(1, D // L):
                    acc = acc + (srows_v[r, pl.ds(j * L, L)]
                                 * trows_v[r, pl.ds(j * L, L)])
                res = jnp.where(lane == e, jnp.sum(acc), res)
            out_v[pl.ds(coff + e0, L)] = res * (1.0 / D)

    pltpu.sync_copy(out_v, out_hbm.at[pl.ds(base, E_PER_W)])


@jax.jit
def kernel(node_embeddings, edge_index):
    idx = edge_index.astype(jnp.int32)
    return _edge_dot(node_embeddings, idx[0], idx[1])


# speedup vs baseline: 3.1428x; 1.1874x over previous
import functools

import jax
import jax.numpy as jnp
from jax import lax
from jax.experimental import pallas as pl
from jax.experimental.pallas import tpu as pltpu
from jax.experimental.pallas import tpu_sc as plsc

N_NODES = 10000
D = 128
E = 320000
NC = 2
NS = 16
NW = NC * NS
E_PER_W = E // NW
C = 40
NCHUNKS = E_PER_W // C
NBUF = 2
L = 16

_mesh = plsc.VectorSubcoreMesh(core_axis_name="c", subcore_axis_name="s")


@functools.partial(
    pl.kernel,
    out_type=jax.ShapeDtypeStruct((E,), jnp.float32),
    mesh=_mesh,
    compiler_params=pltpu.CompilerParams(needs_layout_passes=False),
    scratch_types=[
        pltpu.VMEM((E_PER_W,), jnp.int32),
        pltpu.VMEM((E_PER_W,), jnp.int32),
        pltpu.VMEM((NBUF, C, D), jnp.float32),
        pltpu.VMEM((NBUF, C, D), jnp.float32),
        pltpu.VMEM((E_PER_W,), jnp.float32),
    ] + [pltpu.SemaphoreType.DMA] * NBUF,
)
def _edge_dot(table_hbm, sidx_hbm, tidx_hbm, out_hbm,
              sidx_v, tidx_v, srows_v, trows_v, out_v, *sems):
    wid = lax.axis_index("s") * NC + lax.axis_index("c")
    base = wid * E_PER_W

    pltpu.sync_copy(sidx_hbm.at[pl.ds(base, E_PER_W)], sidx_v)
    pltpu.sync_copy(tidx_hbm.at[pl.ds(base, E_PER_W)], tidx_v)

    lane = lax.iota(jnp.int32, L)

    def start_gathers(ch, b):
        pltpu.async_copy(table_hbm.at[sidx_v.at[pl.ds(ch * C, C)]],
                         srows_v.at[b], sems[b])
        pltpu.async_copy(table_hbm.at[tidx_v.at[pl.ds(ch * C, C)]],
                         trows_v.at[b], sems[b])

    def wait_gathers(b):
        pltpu.make_async_copy(table_hbm.at[pl.ds(0, C)], srows_v.at[b],
                              sems[b]).wait()
        pltpu.make_async_copy(table_hbm.at[pl.ds(0, C)], trows_v.at[b],
                              sems[b]).wait()

    for b in range(NBUF):
        start_gathers(b, b)

    @pl.loop(0, NCHUNKS, step=NBUF)
    def _(ch0):
        for b in range(NBUF):
            ch = ch0 + b
            coff = ch * C
            wait_gathers(b)

            @pl.loop(0, C, step=L)
            def _(e0):
                res = jnp.zeros((L,), jnp.float32)
                for e in range(L):
                    r = e0 + e
                    acc = srows_v[b, r, pl.ds(0, L)] * trows_v[b, r, pl.ds(0, L)]
                    for j in range(1, D // L):
                        acc = acc + (srows_v[b, r, pl.ds(j * L, L)]
                                     * trows_v[b, r, pl.ds(j * L, L)])
                    res = jnp.where(lane == e, jnp.sum(acc), res)
                out_v[pl.ds(coff + e0, L)] = res * (1.0 / D)

            @pl.when(ch + NBUF < NCHUNKS)
            def _():
                start_gathers(ch + NBUF, b)

    pltpu.sync_copy(out_v, out_hbm.at[pl.ds(base, E_PER_W)])


@jax.jit
def kernel(node_embeddings, edge_index):
    idx = edge_index.astype(jnp.int32)
    return _edge_dot(node_embeddings, idx[0], idx[1])


# speedup vs baseline: 9.1584x; 2.9141x over previous
import functools

import jax
import jax.numpy as jnp
from jax import lax
from jax.experimental import pallas as pl
from jax.experimental.pallas import tpu as pltpu
from jax.experimental.pallas import tpu_sc as plsc

N_NODES = 10000
D = 128
E = 320000
NC = 2
NS = 16
NW = NC * NS
E_PER_W = E // NW
C = 80
NCHUNKS = E_PER_W // C
NBUF = 2
NPAIR = (NCHUNKS + NBUF - 1) // NBUF * NBUF
L = 16

_mesh = plsc.VectorSubcoreMesh(core_axis_name="c", subcore_axis_name="s")


@functools.partial(
    pl.kernel,
    out_type=jax.ShapeDtypeStruct((E,), jnp.float32),
    mesh=_mesh,
    compiler_params=pltpu.CompilerParams(needs_layout_passes=False),
    scratch_types=[
        pltpu.VMEM((E_PER_W,), jnp.int32),
        pltpu.VMEM((E_PER_W,), jnp.int32),
        pltpu.VMEM((NBUF, C, D), jnp.float32),
        pltpu.VMEM((NBUF, C, D), jnp.float32),
        pltpu.VMEM((E_PER_W,), jnp.float32),
    ] + [pltpu.SemaphoreType.DMA] * NBUF,
)
def _edge_dot(table_hbm, sidx_hbm, tidx_hbm, out_hbm,
              sidx_v, tidx_v, srows_v, trows_v, out_v, *sems):
    wid = lax.axis_index("s") * NC + lax.axis_index("c")
    base = wid * E_PER_W

    pltpu.sync_copy(sidx_hbm.at[pl.ds(base, E_PER_W)], sidx_v)
    pltpu.sync_copy(tidx_hbm.at[pl.ds(base, E_PER_W)], tidx_v)

    lane = lax.iota(jnp.int32, L)
    m_last = lane == (L - 1)
    zeros_i = jnp.zeros((L,), jnp.int32)

    def start_gathers(ch, b):
        pltpu.async_copy(table_hbm.at[sidx_v.at[pl.ds(ch * C, C)]],
                         srows_v.at[b], sems[b])
        pltpu.async_copy(table_hbm.at[tidx_v.at[pl.ds(ch * C, C)]],
                         trows_v.at[b], sems[b])

    def wait_gathers(b):
        pltpu.make_async_copy(table_hbm.at[pl.ds(0, C)], srows_v.at[b],
                              sems[b]).wait()
        pltpu.make_async_copy(table_hbm.at[pl.ds(0, C)], trows_v.at[b],
                              sems[b]).wait()

    for b in range(NBUF):
        start_gathers(b, b)

    @pl.loop(0, NPAIR, step=NBUF)
    def _(ch0):
        for b in range(NBUF):
            ch = ch0 + b

            @pl.when(ch < NCHUNKS)
            def _():
                coff = ch * C
                wait_gathers(b)

                @plsc.parallel_loop(0, C, unroll=4)
                def _(e):
                    p = [srows_v[b, e, pl.ds(j * L, L)]
                         * trows_v[b, e, pl.ds(j * L, L)]
                         for j in range(D // L)]
                    s0 = (p[0] + p[1]) + (p[2] + p[3])
                    s1 = (p[4] + p[5]) + (p[6] + p[7])
                    csum = jnp.cumsum((s0 + s1) * (1.0 / D))
                    plsc.store_scatter(out_v, [zeros_i + (coff + e)],
                                       csum, mask=m_last)

                @pl.when(ch + NBUF < NCHUNKS)
                def _():
                    start_gathers(ch + NBUF, b)

    pltpu.sync_copy(out_v, out_hbm.at[pl.ds(base, E_PER_W)])


@jax.jit
def kernel(node_embeddings, edge_index):
    idx = edge_index.astype(jnp.int32)
    return _edge_dot(node_embeddings, idx[0], idx[1])


# speedup vs baseline: 9.9525x; 1.0867x over previous
import functools

import jax
import jax.numpy as jnp
from jax import lax
from jax.experimental import pallas as pl
from jax.experimental.pallas import tpu as pltpu
from jax.experimental.pallas import tpu_sc as plsc

N_NODES = 10000
D = 128
E = 320000
NC = 2
NS = 16
NW = NC * NS
E_PER_W = E // NW
C = 80
NCHUNKS = E_PER_W // C
NBUF = 2
NPAIR = (NCHUNKS + NBUF - 1) // NBUF * NBUF
L = 16

_mesh = plsc.VectorSubcoreMesh(core_axis_name="c", subcore_axis_name="s")


@functools.partial(
    pl.kernel,
    out_type=jax.ShapeDtypeStruct((E,), jnp.float32),
    mesh=_mesh,
    compiler_params=pltpu.CompilerParams(needs_layout_passes=False,
                                         use_tc_tiling_on_sc=False),
    scratch_types=[
        pltpu.VMEM((E_PER_W,), jnp.int32),
        pltpu.VMEM((E_PER_W,), jnp.int32),
        pltpu.VMEM((NBUF, C, D // 2), jnp.int32),
        pltpu.VMEM((NBUF, C, D // 2), jnp.int32),
        pltpu.VMEM((E_PER_W,), jnp.float32),
    ] + [pltpu.SemaphoreType.DMA] * NBUF,
)
def _edge_dot(table_hbm, sidx_hbm, tidx_hbm, out_hbm,
              sidx_v, tidx_v, srows_v, trows_v, out_v, *sems):
    wid = lax.axis_index("s") * NC + lax.axis_index("c")
    base = wid * E_PER_W

    pltpu.sync_copy(sidx_hbm.at[pl.ds(base, E_PER_W)], sidx_v)
    pltpu.sync_copy(tidx_hbm.at[pl.ds(base, E_PER_W)], tidx_v)

    lane = lax.iota(jnp.int32, L)
    m_last = lane == (L - 1)
    zeros_i = jnp.zeros((L,), jnp.int32)

    def start_gathers(ch, b):
        pltpu.async_copy(table_hbm.at[sidx_v.at[pl.ds(ch * C, C)]],
                         srows_v.at[b], sems[b])
        pltpu.async_copy(table_hbm.at[tidx_v.at[pl.ds(ch * C, C)]],
                         trows_v.at[b], sems[b])

    def wait_gathers(b):
        pltpu.make_async_copy(table_hbm.at[pl.ds(0, C)], srows_v.at[b],
                              sems[b]).wait()
        pltpu.make_async_copy(table_hbm.at[pl.ds(0, C)], trows_v.at[b],
                              sems[b]).wait()

    for b in range(NBUF):
        start_gathers(b, b)

    @pl.loop(0, NPAIR, step=NBUF)
    def _(ch0):
        for b in range(NBUF):
            ch = ch0 + b

            @pl.when(ch < NCHUNKS)
            def _():
                coff = ch * C
                wait_gathers(b)

                @plsc.parallel_loop(0, C, unroll=4)
                def _(e):
                    p = []
                    for j in range(D // (2 * L)):
                        sj = plsc.bitcast(srows_v[b, e, pl.ds(j * L, L)],
                                          jnp.bfloat16)
                        tj = plsc.bitcast(trows_v[b, e, pl.ds(j * L, L)],
                                          jnp.bfloat16)
                        s_lo, s_hi = plsc.unpack(
                            sj, format=plsc.PackFormat.INTERLEAVED,
                            preferred_element_type=jnp.float32)
                        t_lo, t_hi = plsc.unpack(
                            tj, format=plsc.PackFormat.INTERLEAVED,
                            preferred_element_type=jnp.float32)
                        p.append(s_lo * t_lo)
                        p.append(s_hi * t_hi)
                    s0 = (p[0] + p[1]) + (p[2] + p[3])
                    s1 = (p[4] + p[5]) + (p[6] + p[7])
                    csum = jnp.cumsum((s0 + s1) * (1.0 / D))
                    plsc.store_scatter(out_v, [zeros_i + (coff + e)],
                                       csum, mask=m_last)

                @pl.when(ch + NBUF < NCHUNKS)
                def _():
                    start_gathers(ch + NBUF, b)

    pltpu.sync_copy(out_v, out_hbm.at[pl.ds(base, E_PER_W)])


@jax.jit
def kernel(node_embeddings, edge_index):
    idx = edge_index.astype(jnp.int32)
    table_bf = node_embeddings.astype(jnp.bfloat16).reshape(N_NODES, D // 2, 2)
    table_i32 = jax.lax.bitcast_convert_type(table_bf, jnp.int32)
    return _edge_dot(table_i32, idx[0], idx[1])


# speedup vs baseline: 12.1472x; 1.2205x over previous
import functools

import jax
import jax.numpy as jnp
from jax import lax
from jax.experimental import pallas as pl
from jax.experimental.pallas import tpu as pltpu
from jax.experimental.pallas import tpu_sc as plsc

N_NODES = 10000
D = 128
E = 320000
NC = 2
NS = 16
NW = NC * NS
E_PER_W = E // NW
C = 80
NCHUNKS = E_PER_W // C
NBUF = 4
NPAIR = (NCHUNKS + NBUF - 1) // NBUF * NBUF
L = 16

_mesh = plsc.VectorSubcoreMesh(core_axis_name="c", subcore_axis_name="s")


@functools.partial(
    pl.kernel,
    out_type=jax.ShapeDtypeStruct((E,), jnp.float32),
    mesh=_mesh,
    compiler_params=pltpu.CompilerParams(needs_layout_passes=False,
                                         use_tc_tiling_on_sc=False),
    scratch_types=[
        pltpu.VMEM((E_PER_W,), jnp.int32),
        pltpu.VMEM((E_PER_W,), jnp.int32),
        pltpu.VMEM((NBUF, C, D // 2), jnp.int32),
        pltpu.VMEM((NBUF, C, D // 2), jnp.int32),
        pltpu.VMEM((E_PER_W,), jnp.float32),
    ] + [pltpu.SemaphoreType.DMA] * NBUF,
)
def _edge_dot(table_hbm, sidx_hbm, tidx_hbm, out_hbm,
              sidx_v, tidx_v, srows_v, trows_v, out_v, *sems):
    wid = lax.axis_index("s") * NC + lax.axis_index("c")
    base = wid * E_PER_W

    pltpu.sync_copy(sidx_hbm.at[pl.ds(base, E_PER_W)], sidx_v)
    pltpu.sync_copy(tidx_hbm.at[pl.ds(base, E_PER_W)], tidx_v)

    lane = lax.iota(jnp.int32, L)
    m_last = lane == (L - 1)
    zeros_i = jnp.zeros((L,), jnp.int32)

    def start_gathers(ch, b):
        pltpu.async_copy(table_hbm.at[sidx_v.at[pl.ds(ch * C, C)]],
                         srows_v.at[b], sems[b])
        pltpu.async_copy(table_hbm.at[tidx_v.at[pl.ds(ch * C, C)]],
                         trows_v.at[b], sems[b])

    def wait_gathers(b):
        pltpu.make_async_copy(table_hbm.at[pl.ds(0, C)], srows_v.at[b],
                              sems[b]).wait()
        pltpu.make_async_copy(table_hbm.at[pl.ds(0, C)], trows_v.at[b],
                              sems[b]).wait()

    for b in range(NBUF):
        start_gathers(b, b)

    @pl.loop(0, NPAIR, step=NBUF)
    def _(ch0):
        for b in range(NBUF):
            ch = ch0 + b

            @pl.when(ch < NCHUNKS)
            def _():
                coff = ch * C
                wait_gathers(b)

                @plsc.parallel_loop(0, C, unroll=4)
                def _(e):
                    p = []
                    for j in range(D // (2 * L)):
                        sj = plsc.bitcast(srows_v[b, e, pl.ds(j * L, L)],
                                          jnp.bfloat16)
                        tj = plsc.bitcast(trows_v[b, e, pl.ds(j * L, L)],
                                          jnp.bfloat16)
                        s_lo, s_hi = plsc.unpack(
                            sj, format=plsc.PackFormat.INTERLEAVED,
                            preferred_element_type=jnp.float32)
                        t_lo, t_hi = plsc.unpack(
                            tj, format=plsc.PackFormat.INTERLEAVED,
                            preferred_element_type=jnp.float32)
                        p.append(s_lo * t_lo)
                        p.append(s_hi * t_hi)
                    s0 = (p[0] + p[1]) + (p[2] + p[3])
                    s1 = (p[4] + p[5]) + (p[6] + p[7])
                    csum = jnp.cumsum((s0 + s1) * (1.0 / D))
                    plsc.store_scatter(out_v, [zeros_i + (coff + e)],
                                       csum, mask=m_last)

                @pl.when(ch + NBUF < NCHUNKS)
                def _():
                    start_gathers(ch + NBUF, b)

    pltpu.sync_copy(out_v, out_hbm.at[pl.ds(base, E_PER_W)])


@jax.jit
def kernel(node_embeddings, edge_index):
    idx = edge_index.astype(jnp.int32)
    table_bf = node_embeddings.astype(jnp.bfloat16).reshape(N_NODES, D // 2, 2)
    table_i32 = jax.lax.bitcast_convert_type(table_bf, jnp.int32)
    return _edge_dot(table_i32, idx[0], idx[1])
